# baseline (speedup 1.0000x reference)
import jax
import jax.numpy as jnp
from jax import lax
from jax.experimental import pallas as pl
from jax.experimental.pallas import tpu as pltpu
from jax.experimental.pallas import tpu_sc as plsc

B = 16384
D = 16
V = 1000000
CHUNK = 128
W_IDX = 512
GRP = 16


def _gather_body(sidx_hbm, utab_hbm, itab_hbm,
                 uout_hbm, iout_hbm,
                 idx_v, grp0_v, grp1_v, du_v, di_v,
                 *sems16):
    wid = lax.axis_index("s") * 2 + lax.axis_index("c")

    pltpu.sync_copy(sidx_hbm.at[pl.ds(pl.multiple_of(wid * 8, 8), 8)], idx_v)

    grps = (grp0_v, grp1_v)
    sems = (sems16[0:8], sems16[8:16])
    n_grp = W_IDX // GRP

    def fire(tab, r0, g, buf):
        p = g * GRP
        v = idx_v[r0 + lax.div(p, 128), pl.ds(lax.rem(p, 128), 16)]
        for j in range(GRP):
            off = pl.multiple_of(
                lax.shift_right_logical(v[j], 7) * 128, 128)
            pltpu.make_async_copy(
                tab.at[pl.ds(0, 8), pl.ds(off, 128)],
                grps[buf].at[pl.ds(0, 8), pl.ds(j * 128, 128)],
                sems[buf][j % 4]).start()
            pltpu.make_async_copy(
                tab.at[pl.ds(8, 8), pl.ds(off, 128)],
                grps[buf].at[pl.ds(8, 8), pl.ds(j * 128, 128)],
                sems[buf][4 + j % 4]).start()

    def drain_extract(tab, dst, r0, g, buf):
        for q in range(8):
            pltpu.make_async_copy(
                tab.at[pl.ds(0, 8), pl.ds(0, GRP * 32)],
                grps[buf].at[pl.ds(0, 8), pl.ds(0, GRP * 32)],
                sems[buf][q]).wait()
        p = g * GRP
        lv = idx_v[r0 + lax.div(p, 128), pl.ds(lax.rem(p, 128), 16)] & 127
        colv = lax.iota(jnp.int32, 16) * 128 + lv
        bv = lax.iota(jnp.int32, 16) + p

        def dbody(d, _):
            dv = jnp.zeros((16,), jnp.int32) + d
            vals = plsc.load_gather(grps[buf], [dv, colv])
            plsc.store_scatter(dst, [dv, bv], vals)
            return 0

        lax.fori_loop(0, D, dbody, 0)

    fire(utab_hbm, 0, 0, 0)

    def body(g, _):
        fire(itab_hbm, 4, g, 1)
        drain_extract(utab_hbm, du_v, 0, g, 0)
        fire(utab_hbm, 0, g + 1, 0)
        drain_extract(itab_hbm, di_v, 4, g, 1)
        return 0

    lax.fori_loop(0, n_grp - 1, body, 0)
    g_last = n_grp - 1
    fire(itab_hbm, 4, g_last, 1)
    drain_extract(utab_hbm, du_v, 0, g_last, 0)
    drain_extract(itab_hbm, di_v, 4, g_last, 1)

    obase = pl.multiple_of(wid * W_IDX, 128)
    pltpu.sync_copy(du_v, uout_hbm.at[:, pl.ds(obase, W_IDX)])
    pltpu.sync_copy(di_v, iout_hbm.at[:, pl.ds(obase, W_IDX)])


def _sc_gather(sidx, utab_t, itab_t):
    mesh = plsc.VectorSubcoreMesh(core_axis_name="c", subcore_axis_name="s")
    f = pl.kernel(
        _gather_body,
        mesh=mesh,
        compiler_params=pltpu.CompilerParams(needs_layout_passes=False),
        out_type=[
            jax.ShapeDtypeStruct((D, B), jnp.float32),
            jax.ShapeDtypeStruct((D, B), jnp.float32),
        ],
        scratch_types=[
            pltpu.VMEM((8, CHUNK), jnp.int32),
            pltpu.VMEM((D, GRP * 128), jnp.float32),
            pltpu.VMEM((D, GRP * 128), jnp.float32),
            pltpu.VMEM((D, W_IDX), jnp.float32),
            pltpu.VMEM((D, W_IDX), jnp.float32),
        ] + [pltpu.SemaphoreType.DMA] * 16,
    )
    return f(sidx, utab_t, itab_t)


def _mlp_body(u_ref, i_ref, w1u_ref, w1i_ref, b1_ref, w2_ref, b2_ref,
              w3_ref, b3_ref, out_ref):
    x = (jnp.dot(w1u_ref[...], u_ref[...], preferred_element_type=jnp.float32)
         + jnp.dot(w1i_ref[...], i_ref[...], preferred_element_type=jnp.float32)
         + b1_ref[...])
    h1 = jnp.maximum(x, 0.0)
    h2 = jnp.maximum(
        jnp.dot(w2_ref[...], h1, preferred_element_type=jnp.float32)
        + b2_ref[...], 0.0)
    out_ref[...] = (
        jnp.dot(w3_ref[...], h2, preferred_element_type=jnp.float32)
        + b3_ref[...])


def _tc_mlp(u_t, i_t, w1u, w1i, b1, w2, b2, w3, b3):
    return pl.pallas_call(
        _mlp_body,
        out_shape=jax.ShapeDtypeStruct((1, B), jnp.float32),
    )(u_t, i_t, w1u, w1i, b1, w2, b2, w3, b3)


def kernel(user, item, user_table, item_table, W1, b1, W2, b2, W3, b3):
    uidx = user.astype(jnp.int32).reshape(32, 4, CHUNK)
    iidx = item.astype(jnp.int32).reshape(32, 4, CHUNK)
    sidx = jnp.concatenate([uidx, iidx], axis=1).reshape(256, CHUNK)
    u_t, i_t = _sc_gather(sidx, user_table.T, item_table.T)
    out_t = _tc_mlp(u_t, i_t, W1[:, :D], W1[:, D:], b1.reshape(64, 1),
                    W2, b2.reshape(32, 1), W3, b3.reshape(1, 1))
    return out_t.reshape(B, 1)


# speedup vs baseline: 1.2191x; 1.2191x over previous
import jax
import jax.numpy as jnp
from jax import lax
from jax.experimental import pallas as pl
from jax.experimental.pallas import tpu as pltpu
from jax.experimental.pallas import tpu_sc as plsc

B = 16384
D = 16
V = 1000000
CHUNK = 128
W_IDX = 512
GRP = 16


def _gather_body(sidx_hbm, utab_hbm, itab_hbm,
                 uout_hbm, iout_hbm,
                 idx_v, grp0_v, grp1_v, du_v, di_v,
                 sem0a, sem0b, sem0c, sem0d, sem1a, sem1b, sem1c, sem1d):
    wid = lax.axis_index("s") * 2 + lax.axis_index("c")

    pltpu.sync_copy(sidx_hbm.at[pl.ds(pl.multiple_of(wid * 8, 8), 8)], idx_v)

    grps = (grp0_v, grp1_v)
    sems = ((sem0a, sem0b, sem0c, sem0d), (sem1a, sem1b, sem1c, sem1d))
    n_grp = W_IDX // GRP

    def fire(tab, r0, g, buf):
        p = g * GRP
        v = idx_v[r0 + lax.div(p, 128), pl.ds(lax.rem(p, 128), 16)]
        for j in range(GRP):
            off = pl.multiple_of(
                lax.shift_right_logical(v[j], 7) * 128, 128)
            pltpu.make_async_copy(
                tab.at[pl.ds(0, 8), pl.ds(off, 128)],
                grps[buf].at[pl.ds(0, 8), pl.ds(j * 128, 128)],
                sems[buf][j % 2]).start()
            pltpu.make_async_copy(
                tab.at[pl.ds(8, 8), pl.ds(off, 128)],
                grps[buf].at[pl.ds(8, 8), pl.ds(j * 128, 128)],
                sems[buf][2 + j % 2]).start()

    def drain_extract(tab, dst, r0, g, buf):
        for q in range(4):
            pltpu.make_async_copy(
                tab.at[pl.ds(0, 8), pl.ds(0, GRP * 64)],
                grps[buf].at[pl.ds(0, 8), pl.ds(0, GRP * 64)],
                sems[buf][q]).wait()
        p = g * GRP
        lv = idx_v[r0 + lax.div(p, 128), pl.ds(lax.rem(p, 128), 16)] & 127
        colv = lax.iota(jnp.int32, 16) * 128 + lv
        bv = lax.iota(jnp.int32, 16) + p

        def dbody(d, _):
            dv = jnp.zeros((16,), jnp.int32) + d
            vals = plsc.load_gather(grps[buf], [dv, colv])
            plsc.store_scatter(dst, [dv, bv], vals)
            return 0

        lax.fori_loop(0, D, dbody, 0)

    fire(utab_hbm, 0, 0, 0)

    def body(g, _):
        fire(itab_hbm, 4, g, 1)
        drain_extract(utab_hbm, du_v, 0, g, 0)
        fire(utab_hbm, 0, g + 1, 0)
        drain_extract(itab_hbm, di_v, 4, g, 1)
        return 0

    lax.fori_loop(0, n_grp - 1, body, 0)
    g_last = n_grp - 1
    fire(itab_hbm, 4, g_last, 1)
    drain_extract(utab_hbm, du_v, 0, g_last, 0)
    drain_extract(itab_hbm, di_v, 4, g_last, 1)

    obase = pl.multiple_of(wid * W_IDX, 128)
    pltpu.sync_copy(du_v, uout_hbm.at[:, pl.ds(obase, W_IDX)])
    pltpu.sync_copy(di_v, iout_hbm.at[:, pl.ds(obase, W_IDX)])


def _sc_gather(sidx, utab_t, itab_t):
    mesh = plsc.VectorSubcoreMesh(core_axis_name="c", subcore_axis_name="s")
    f = pl.kernel(
        _gather_body,
        mesh=mesh,
        compiler_params=pltpu.CompilerParams(needs_layout_passes=False),
        out_type=[
            jax.ShapeDtypeStruct((D, B), jnp.float32),
            jax.ShapeDtypeStruct((D, B), jnp.float32),
        ],
        scratch_types=[
            pltpu.VMEM((8, CHUNK), jnp.int32),
            pltpu.VMEM((D, GRP * 128), jnp.float32),
            pltpu.VMEM((D, GRP * 128), jnp.float32),
            pltpu.VMEM((D, W_IDX), jnp.float32),
            pltpu.VMEM((D, W_IDX), jnp.float32),
        ] + [pltpu.SemaphoreType.DMA] * 8,
    )
    return f(sidx, utab_t, itab_t)


def _mlp_body(u_ref, i_ref, w1u_ref, w1i_ref, b1_ref, w2_ref, b2_ref,
              w3_ref, b3_ref, out_ref):
    x = (jnp.dot(w1u_ref[...], u_ref[...], preferred_element_type=jnp.float32)
         + jnp.dot(w1i_ref[...], i_ref[...], preferred_element_type=jnp.float32)
         + b1_ref[...])
    h1 = jnp.maximum(x, 0.0)
    h2 = jnp.maximum(
        jnp.dot(w2_ref[...], h1, preferred_element_type=jnp.float32)
        + b2_ref[...], 0.0)
    out_ref[...] = (
        jnp.dot(w3_ref[...], h2, preferred_element_type=jnp.float32)
        + b3_ref[...])


def _tc_mlp(u_t, i_t, w1u, w1i, b1, w2, b2, w3, b3):
    return pl.pallas_call(
        _mlp_body,
        out_shape=jax.ShapeDtypeStruct((1, B), jnp.float32),
    )(u_t, i_t, w1u, w1i, b1, w2, b2, w3, b3)


def kernel(user, item, user_table, item_table, W1, b1, W2, b2, W3, b3):
    uidx = user.astype(jnp.int32).reshape(32, 4, CHUNK)
    iidx = item.astype(jnp.int32).reshape(32, 4, CHUNK)
    sidx = jnp.concatenate([uidx, iidx], axis=1).reshape(256, CHUNK)
    u_t, i_t = _sc_gather(sidx, user_table.T, item_table.T)
    out_t = _tc_mlp(u_t, i_t, W1[:, :D], W1[:, D:], b1.reshape(64, 1),
                    W2, b2.reshape(32, 1), W3, b3.reshape(1, 1))
    return out_t.reshape(B, 1)
